# Initial kernel scaffold; baseline (speedup 1.0000x reference)
import functools

import jax
import jax.numpy as jnp
from jax import lax
from jax.experimental import pallas as pl
from jax.experimental.pallas import tpu as pltpu
from jax.experimental.pallas import tpu_sc as plsc

PAD_IDX = 0

_NUM_CORES = 2
_NUM_SUBCORES = 16
_NW = _NUM_CORES * _NUM_SUBCORES

_CHUNK = 128


def _mask_body(idx_ref, mask_ref):
    mask_ref[...] = (idx_ref[...] != PAD_IDX).astype(jnp.int32)


@functools.lru_cache(maxsize=None)
def _make_gather(n_rows, vocab, dim):
    assert n_rows % (_NW * _CHUNK) == 0
    chunks_per_w = n_rows // (_NW * _CHUNK)

    mesh = plsc.VectorSubcoreMesh(
        core_axis_name="c",
        subcore_axis_name="s",
        num_cores=_NUM_CORES,
        num_subcores=_NUM_SUBCORES,
    )

    @functools.partial(
        pl.kernel,
        mesh=mesh,
        out_type=jax.ShapeDtypeStruct((n_rows, dim), jnp.float32),
        scratch_types=[
            pltpu.VMEM((chunks_per_w, _CHUNK), jnp.int32),
            pltpu.VMEM((_CHUNK, dim), jnp.float32),
            pltpu.SemaphoreType.DMA,
        ],
    )
    def gather_kernel(idx_hbm, table_hbm, out_hbm, idx_v, rows_v, sem):
        wid = lax.axis_index("s") * _NUM_CORES + lax.axis_index("c")
        base_chunk = wid * chunks_per_w
        pltpu.sync_copy(idx_hbm.at[pl.ds(base_chunk, chunks_per_w)], idx_v)

        def body(c, carry):
            pltpu.async_copy(table_hbm.at[idx_v.at[c]], rows_v, sem).wait()
            pltpu.sync_copy(
                rows_v, out_hbm.at[pl.ds((base_chunk + c) * _CHUNK, _CHUNK)]
            )
            return carry

        lax.fori_loop(0, chunks_per_w, body, 0)

    return gather_kernel


def kernel(indices, table):
    batch, seq = indices.shape
    vocab, dim = table.shape
    n_rows = batch * seq
    idx2d = indices.reshape(n_rows // _CHUNK, _CHUNK)

    encoded_flat = _make_gather(n_rows, vocab, dim)(idx2d, table)
    encoded = encoded_flat.reshape(batch, seq, dim)

    mask = pl.pallas_call(
        _mask_body,
        out_shape=jax.ShapeDtypeStruct((batch, seq), jnp.int32),
    )(indices)
    return encoded, mask


# speedup vs baseline: 2.9457x; 2.9457x over previous
import functools

import jax
import jax.numpy as jnp
from jax import lax
from jax.experimental import pallas as pl
from jax.experimental.pallas import tpu as pltpu
from jax.experimental.pallas import tpu_sc as plsc

PAD_IDX = 0

_NUM_CORES = 2
_NUM_SUBCORES = 16
_NW = _NUM_CORES * _NUM_SUBCORES

_CHUNK = 128


def _mask_body(idx_ref, mask_ref):
    mask_ref[...] = (idx_ref[...] != PAD_IDX).astype(jnp.int32)


@functools.lru_cache(maxsize=None)
def _make_gather(n_rows, vocab, dim):
    assert n_rows % (_NW * _CHUNK) == 0
    chunks_per_w = n_rows // (_NW * _CHUNK)

    mesh = plsc.VectorSubcoreMesh(
        core_axis_name="c",
        subcore_axis_name="s",
        num_cores=_NUM_CORES,
        num_subcores=_NUM_SUBCORES,
    )

    @functools.partial(
        pl.kernel,
        mesh=mesh,
        out_type=jax.ShapeDtypeStruct((n_rows, dim), jnp.float32),
        scratch_types=[
            pltpu.VMEM((chunks_per_w, _CHUNK), jnp.int32),
            pltpu.VMEM((_CHUNK, dim), jnp.float32),
            pltpu.SemaphoreType.DMA,
        ],
    )
    def gather_kernel(idx_hbm, table_hbm, out_hbm, idx_v, rows_v, sem):
        wid = lax.axis_index("s") * _NUM_CORES + lax.axis_index("c")
        base_chunk = wid * chunks_per_w
        pltpu.sync_copy(idx_hbm.at[wid], idx_v)

        def body(c, carry):
            pltpu.async_copy(table_hbm.at[idx_v.at[c]], rows_v, sem).wait()
            pltpu.sync_copy(
                rows_v, out_hbm.at[pl.ds((base_chunk + c) * _CHUNK, _CHUNK)]
            )
            return carry

        lax.fori_loop(0, chunks_per_w, body, 0)

    return gather_kernel


def kernel(indices, table):
    batch, seq = indices.shape
    vocab, dim = table.shape
    n_rows = batch * seq
    idx2d = indices.reshape(_NW, n_rows // (_NW * _CHUNK), _CHUNK)

    encoded_flat = _make_gather(n_rows, vocab, dim)(idx2d, table)
    encoded = encoded_flat.reshape(batch, seq, dim)

    mask = pl.pallas_call(
        _mask_body,
        out_shape=jax.ShapeDtypeStruct((batch, seq), jnp.int32),
    )(indices)
    return encoded, mask


# speedup vs baseline: 3.3344x; 1.1320x over previous
import functools

import jax
import jax.numpy as jnp
from jax import lax
from jax.experimental import pallas as pl
from jax.experimental.pallas import tpu as pltpu
from jax.experimental.pallas import tpu_sc as plsc

PAD_IDX = 0

_NUM_CORES = 2
_NUM_SUBCORES = 16
_NW = _NUM_CORES * _NUM_SUBCORES

_CHUNK = 128


def _mask_body(idx_ref, mask_ref):
    mask_ref[...] = (idx_ref[...] != PAD_IDX).astype(jnp.int32)


_NBUF = 5


@functools.lru_cache(maxsize=None)
def _make_gather(n_rows, vocab, dim):
    assert n_rows % (_NW * _CHUNK) == 0
    chunks_per_w = n_rows // (_NW * _CHUNK)
    assert chunks_per_w % _NBUF == 0 and chunks_per_w >= 2 * _NBUF
    main_iters = chunks_per_w // _NBUF - 1

    mesh = plsc.VectorSubcoreMesh(
        core_axis_name="c",
        subcore_axis_name="s",
        num_cores=_NUM_CORES,
        num_subcores=_NUM_SUBCORES,
    )

    @functools.partial(
        pl.kernel,
        mesh=mesh,
        out_type=jax.ShapeDtypeStruct((n_rows, dim), jnp.float32),
        scratch_types=[
            pltpu.VMEM((chunks_per_w, _CHUNK), jnp.int32),
            pltpu.VMEM((_NBUF, _CHUNK, dim), jnp.float32),
            [pltpu.SemaphoreType.DMA] * _NBUF,
            [pltpu.SemaphoreType.DMA] * _NBUF,
        ],
    )
    def gather_kernel(idx_hbm, table_hbm, out_hbm, idx_v, rows_v, gsems, wsems):
        wid = lax.axis_index("s") * _NUM_CORES + lax.axis_index("c")
        base_chunk = wid * chunks_per_w
        pltpu.sync_copy(idx_hbm.at[wid], idx_v)

        def g_copy(b, c):
            return pltpu.make_async_copy(
                table_hbm.at[idx_v.at[c]], rows_v.at[b], gsems[b]
            )

        def w_copy(b, c):
            return pltpu.make_async_copy(
                rows_v.at[b],
                out_hbm.at[pl.ds((base_chunk + c) * _CHUNK, _CHUNK)],
                wsems[b],
            )

        for b in range(_NBUF):
            g_copy(b, b).start()

        def body(j, carry):
            g = j * _NBUF
            for b in range(_NBUF):
                c = g + b
                g_copy(b, c).wait()
                w_copy(b, c).start()
                w_copy(b, c).wait()
                g_copy(b, c + _NBUF).start()
            return carry

        lax.fori_loop(0, main_iters, body, 0)

        tail = chunks_per_w - _NBUF
        for b in range(_NBUF):
            g_copy(b, tail + b).wait()
            w_copy(b, tail + b).start()
        for b in range(_NBUF):
            w_copy(b, tail + b).wait()

    return gather_kernel


def kernel(indices, table):
    batch, seq = indices.shape
    vocab, dim = table.shape
    n_rows = batch * seq
    idx2d = indices.reshape(_NW, n_rows // (_NW * _CHUNK), _CHUNK)

    encoded_flat = _make_gather(n_rows, vocab, dim)(idx2d, table)
    encoded = encoded_flat.reshape(batch, seq, dim)

    mask = pl.pallas_call(
        _mask_body,
        out_shape=jax.ShapeDtypeStruct((batch, seq), jnp.int32),
    )(indices)
    return encoded, mask


# speedup vs baseline: 5.8591x; 1.7572x over previous
import functools

import jax
import jax.numpy as jnp
from jax import lax
from jax.experimental import pallas as pl
from jax.experimental.pallas import tpu as pltpu
from jax.experimental.pallas import tpu_sc as plsc

PAD_IDX = 0

_NUM_CORES = 2
_NUM_SUBCORES = 16
_NW = _NUM_CORES * _NUM_SUBCORES

_CB = 2
_NBUF = 4


def _mask_body(idx_ref, mask_ref):
    mask_ref[...] = (idx_ref[...] != PAD_IDX).astype(jnp.int32)


@functools.lru_cache(maxsize=None)
def _make_gather(batch, seq, vocab, dim):
    assert batch % (_NW * _CB) == 0
    rows_per_w = batch // _NW
    chunks_per_w = rows_per_w // _CB
    chunk_idx = _CB * seq
    assert chunks_per_w % _NBUF == 0 and chunks_per_w >= 2 * _NBUF
    main_iters = chunks_per_w // _NBUF - 1

    mesh = plsc.VectorSubcoreMesh(
        core_axis_name="c",
        subcore_axis_name="s",
        num_cores=_NUM_CORES,
        num_subcores=_NUM_SUBCORES,
    )

    @functools.partial(
        pl.kernel,
        mesh=mesh,
        out_type=jax.ShapeDtypeStruct((batch, seq, dim), jnp.float32),
        scratch_types=[
            pltpu.VMEM((chunks_per_w, chunk_idx), jnp.int32),
            pltpu.VMEM((_NBUF, chunk_idx, dim), jnp.float32),
            [pltpu.SemaphoreType.DMA] * _NBUF,
            [pltpu.SemaphoreType.DMA] * _NBUF,
        ],
    )
    def gather_kernel(idx_hbm, table_hbm, out_hbm, idx_v, rows_v, gsems, wsems):
        wid = lax.axis_index("s") * _NUM_CORES + lax.axis_index("c")
        row0 = wid * rows_per_w
        pltpu.sync_copy(idx_hbm.at[wid], idx_v)

        def g_copy(b, c):
            return pltpu.make_async_copy(
                table_hbm.at[idx_v.at[c]], rows_v.at[b], gsems[b]
            )

        def w_copies(b, c):
            return [
                pltpu.make_async_copy(
                    rows_v.at[b].at[pl.ds(k * seq, seq)],
                    out_hbm.at[row0 + c * _CB + k],
                    wsems[b],
                )
                for k in range(_CB)
            ]

        for b in range(_NBUF):
            g_copy(b, b).start()

        def body(j, carry):
            g = j * _NBUF
            for b in range(_NBUF):
                c = g + b
                g_copy(b, c).wait()
                ws = w_copies(b, c)
                for w in ws:
                    w.start()
                for w in ws:
                    w.wait()
                g_copy(b, c + _NBUF).start()
            return carry

        lax.fori_loop(0, main_iters, body, 0)

        tail = chunks_per_w - _NBUF
        tail_ws = []
        for b in range(_NBUF):
            g_copy(b, tail + b).wait()
            ws = w_copies(b, tail + b)
            for w in ws:
                w.start()
            tail_ws.append(ws)
        for ws in tail_ws:
            for w in ws:
                w.wait()

    return gather_kernel


def kernel(indices, table):
    batch, seq = indices.shape
    vocab, dim = table.shape
    chunks_per_w = batch // (_NW * _CB)
    idx3d = indices.reshape(_NW, chunks_per_w, _CB * seq)

    encoded = _make_gather(batch, seq, vocab, dim)(idx3d, table)

    mask = pl.pallas_call(
        _mask_body,
        out_shape=jax.ShapeDtypeStruct((batch, seq), jnp.int32),
    )(indices)
    return encoded, mask


# speedup vs baseline: 10.2609x; 1.7513x over previous
import functools

import jax
import jax.numpy as jnp
from jax import lax
from jax.experimental import pallas as pl
from jax.experimental.pallas import tpu as pltpu
from jax.experimental.pallas import tpu_sc as plsc

PAD_IDX = 0

_NUM_CORES = 2
_NUM_SUBCORES = 16
_NW = _NUM_CORES * _NUM_SUBCORES

_NBUF = 5


def _mask_body(idx_ref, mask_ref):
    mask_ref[...] = (idx_ref[...] != PAD_IDX).astype(jnp.int32)


@functools.lru_cache(maxsize=None)
def _make_gather(batch, seq, vocab, dim):
    assert batch % _NW == 0
    rows_per_w = batch // _NW
    assert seq % _NBUF == 0 and seq >= 2 * _NBUF
    main_iters = seq // _NBUF - 1

    mesh = plsc.VectorSubcoreMesh(
        core_axis_name="c",
        subcore_axis_name="s",
        num_cores=_NUM_CORES,
        num_subcores=_NUM_SUBCORES,
    )

    @functools.partial(
        pl.kernel,
        mesh=mesh,
        out_type=jax.ShapeDtypeStruct((seq, batch, dim), jnp.float32),
        scratch_types=[
            pltpu.VMEM((seq, rows_per_w), jnp.int32),
            pltpu.VMEM((_NBUF, rows_per_w, dim), jnp.float32),
            [pltpu.SemaphoreType.DMA] * _NBUF,
            [pltpu.SemaphoreType.DMA] * _NBUF,
        ],
    )
    def gather_kernel(idx_hbm, table_hbm, out_hbm, idx_v, rows_v, gsems, wsems):
        wid = lax.axis_index("s") * _NUM_CORES + lax.axis_index("c")
        row0 = wid * rows_per_w
        pltpu.sync_copy(idx_hbm.at[wid], idx_v)

        def g_copy(b, c):
            return pltpu.make_async_copy(
                table_hbm.at[idx_v.at[c]], rows_v.at[b], gsems[b]
            )

        def w_copy(b, c):
            return pltpu.make_async_copy(
                rows_v.at[b],
                out_hbm.at[c].at[pl.ds(row0, rows_per_w)],
                wsems[b],
            )

        for b in range(_NBUF):
            g_copy(b, b).start()

        def body(j, carry):
            g = j * _NBUF
            for b in range(_NBUF):
                c = g + b
                g_copy(b, c).wait()
                w_copy(b, c).start()
                w_copy(b, c).wait()
                g_copy(b, c + _NBUF).start()
            return carry

        lax.fori_loop(0, main_iters, body, 0)

        tail = seq - _NBUF
        for b in range(_NBUF):
            g_copy(b, tail + b).wait()
            w_copy(b, tail + b).start()
        for b in range(_NBUF):
            w_copy(b, tail + b).wait()

    return gather_kernel


def kernel(indices, table):
    batch, seq = indices.shape
    vocab, dim = table.shape
    rows_per_w = batch // _NW
    idx3d = indices.T.reshape(seq, _NW, rows_per_w).transpose(1, 0, 2)

    out_sbd = _make_gather(batch, seq, vocab, dim)(idx3d, table)
    encoded = out_sbd.transpose(1, 0, 2)

    mask = pl.pallas_call(
        _mask_body,
        out_shape=jax.ShapeDtypeStruct((batch, seq), jnp.int32),
    )(indices)
    return encoded, mask
